# baseline (speedup 1.0000x reference)
import functools

import numpy as np

import jax
import jax.numpy as jnp
from jax import lax
from jax.experimental import pallas as pl
from jax.experimental.pallas import tpu as pltpu
from jax.experimental.pallas import tpu_sc as plsc

N = 10000
E = 160000
D = 256
DE = 16
B = 64

H = D // 2
NS = 16
EPT = E // NS
C = 40
NCHUNK = EPT // C
N2 = 10240
RPT = N2 // NS
RW = 40

HW = H // 2
_COLS_LO = tuple(32 * g + i for g in range(H // 32) for i in range(16))
_COLS_HI = tuple(32 * g + 16 + i for g in range(H // 32) for i in range(16))

_HIGH = lax.Precision.HIGHEST
_HI16 = np.int32(-65536)


def _pack_i32(lo_f32, hi_f32):
    lo = lax.bitcast_convert_type(lo_f32.astype(jnp.bfloat16), jnp.int16)
    hi = lax.bitcast_convert_type(hi_f32.astype(jnp.bfloat16), jnp.int16)
    return ((lo.astype(jnp.int32) & 0xFFFF)
            | (hi.astype(jnp.int32) << 16))


def _k1_body(ea_ref, wll_ref, wlh_ref, wrl_ref, wrh_ref, ol_ref, or_ref):
    a = ea_ref[...]
    ol_ref[...] = _pack_i32(lax.dot(a, wll_ref[...], precision=_HIGH),
                            lax.dot(a, wlh_ref[...], precision=_HIGH))
    or_ref[...] = _pack_i32(lax.dot(a, wrl_ref[...], precision=_HIGH),
                            lax.dot(a, wrh_ref[...], precision=_HIGH))


def _edge_matmul(ea, wll, wlh, wrl, wrh):
    EB = 2000
    wspec = pl.BlockSpec((DE, HW), lambda i: (0, 0))
    return pl.pallas_call(
        _k1_body,
        grid=(E // EB,),
        in_specs=[pl.BlockSpec((EB, DE), lambda i: (i, 0)),
                  wspec, wspec, wspec, wspec],
        out_specs=[
            pl.BlockSpec((EB, HW), lambda i: (i, 0)),
            pl.BlockSpec((EB, HW), lambda i: (i, 0)),
        ],
        out_shape=[jax.ShapeDtypeStruct((E, HW), jnp.int32)] * 2,
    )(ea, wll, wlh, wrl, wrh)


def _sc_body(xl, xr, src, dst, el, er, outl, outr,
             idxs0, bufx0, bufe0, bufm0, semi0, seml0, sems0,
             idxs1, bufx1, bufe1, bufm1, semi1, seml1, sems1,
             d0, d1, d2, d3, spm):
    c = lax.axis_index("c")
    s = lax.axis_index("s")
    zero16 = jnp.zeros((16,), jnp.float32)
    sets = ((idxs0, bufx0, bufe0, bufm0, semi0, seml0, sems0),
            (idxs1, bufx1, bufe1, bufm1, semi1, seml1, sems1))
    ring = (d0, d1, d2, d3)

    def _ebase(k):
        return pl.multiple_of(s * EPT + k * C, 8)

    def _stage(k, st, dd):
        eb = _ebase(k)
        pltpu.async_copy(src.at[pl.ds(eb, C)], st[0], st[4])
        pltpu.async_copy(dst.at[pl.ds(eb, C)], dd, st[4])

    def _launch(k, st):
        pltpu.make_async_copy(src.at[pl.ds(0, C)], st[0], st[4]).wait()
        pltpu.make_async_copy(dst.at[pl.ds(0, C)], st[0], st[4]).wait()
        ew_off = pl.multiple_of((s * EPT + k * C) * HW, 128)

        @pl.when(c == 0)
        def _():
            pltpu.async_copy(el.at[pl.ds(ew_off, C * HW)], st[2], st[5])
            pltpu.async_copy(xl.at[st[0]], st[1], st[5])

        @pl.when(c == 1)
        def _():
            pltpu.async_copy(er.at[pl.ds(ew_off, C * HW)], st[2], st[5])
            pltpu.async_copy(xr.at[st[0]], st[1], st[5])

    def _wait_load(st):
        pltpu.make_async_copy(el.at[pl.ds(0, C * HW)], st[2], st[5]).wait()
        pltpu.make_async_copy(xl.at[st[0]], st[1], st[5]).wait()

    def _wait_scatter(st, dd):
        pltpu.make_async_copy(st[3], spm.at[dd], st[6]).wait()

    def _zrow(i, carry):
        for p in range(H // 16):
            bufm0[i, pl.ds(p * 16, 16)] = zero16
        return carry
    lax.fori_loop(0, RW, _zrow, 0)
    base_row = s * RPT
    for j in range(RPT // RW):
        pltpu.sync_copy(bufm0, spm.at[pl.ds(base_row + j * RW, RW)])
    plsc.subcore_barrier()

    def _step(k, st_cur, st_nxt, d_cur, d_stage, first):
        if not first:
            _wait_scatter(st_cur, d_cur)
        _wait_load(st_cur)

        @pl.when(k + 1 < NCHUNK)
        def _():
            _launch(k + 1, st_nxt)

        @pl.when(k + 2 < NCHUNK)
        def _():
            _stage(k + 2, st_cur, d_stage)

        bufx, bufe, bufm = st_cur[1], st_cur[2], st_cur[3]

        def _edge(e2, cc):
            for u in range(2):
                e = e2 * 2 + u
                for g in range(H // 32):
                    we = bufe[pl.ds(e * HW + 16 * g, 16)]
                    ea = lax.bitcast_convert_type(we << 16, jnp.float32)
                    eb = lax.bitcast_convert_type(we & _HI16, jnp.float32)
                    xa = bufx[e, pl.ds(32 * g, 16)]
                    xb = bufx[e, pl.ds(32 * g + 16, 16)]
                    bufm[e, pl.ds(32 * g, 16)] = jnp.maximum(xa + ea, 0.0)
                    bufm[e, pl.ds(32 * g + 16, 16)] = jnp.maximum(xb + eb, 0.0)
            return cc
        lax.fori_loop(0, C // 2, _edge, 0)
        pltpu.async_copy(bufm, spm.at[d_cur], st_cur[6], add=True)

    _stage(0, sets[0], ring[0])
    _stage(1, sets[1], ring[1])
    _launch(0, sets[0])
    _step(0, sets[0], sets[1], ring[0], ring[2], True)
    _step(1, sets[1], sets[0], ring[1], ring[3], True)

    def _quad(j, carry):
        k = 4 * j + 2
        _step(k, sets[0], sets[1], ring[2], ring[0], False)
        _step(k + 1, sets[1], sets[0], ring[3], ring[1], False)
        _step(k + 2, sets[0], sets[1], ring[0], ring[2], False)
        _step(k + 3, sets[1], sets[0], ring[1], ring[3], False)
        return carry
    nquad = (NCHUNK - 2) // 4
    lax.fori_loop(0, nquad, _quad, 0)
    for k in range(2 + 4 * nquad, NCHUNK):
        _step(k, sets[k % 2], sets[(k + 1) % 2],
              ring[k % 4], ring[(k + 2) % 4], False)
    for k in (NCHUNK - 2, NCHUNK - 1):
        _wait_scatter(sets[k % 2], ring[k % 4])
    plsc.subcore_barrier()

    for j in range(RPT // RW):
        rs = base_row + j * RW
        pltpu.sync_copy(spm.at[pl.ds(rs, RW)], bufm0)

        @pl.when(c == 0)
        def _():
            pltpu.sync_copy(bufm0, outl.at[pl.ds(rs, RW)])

        @pl.when(c == 1)
        def _():
            pltpu.sync_copy(bufm0, outr.at[pl.ds(rs, RW)])


@functools.lru_cache(maxsize=None)
def _build_sc_kernel():
    return pl.kernel(
        _sc_body,
        mesh=plsc.VectorSubcoreMesh(
            core_axis_name="c", subcore_axis_name="s", num_cores=2,
            num_subcores=NS),
        out_type=(jax.ShapeDtypeStruct((N2, H), jnp.float32),
                  jax.ShapeDtypeStruct((N2, H), jnp.float32)),
        scratch_types=(
            [pltpu.VMEM((C,), jnp.int32),
             pltpu.VMEM((C, H), jnp.float32),
             pltpu.VMEM((C * HW,), jnp.int32),
             pltpu.VMEM((C, H), jnp.float32),
             pltpu.SemaphoreType.DMA,
             pltpu.SemaphoreType.DMA,
             pltpu.SemaphoreType.DMA] * 2
            + [pltpu.VMEM((C,), jnp.int32)] * 4
            + [pltpu.VMEM_SHARED((N2, H), jnp.float32)]),
    )


def _edge_aggregate(xl, xr, src, dst, el, er):
    return _build_sc_kernel()(xl, xr, src, dst, el, er)


def _k0_body(bp_ref, pri_ref, thr_ref):
    bi = bp_ref[...]
    starts = jnp.stack(
        [jnp.sum((bi < g).astype(jnp.float32)) for g in range(B)])
    thr_ref[...] = (starts + pri_ref[0, :].astype(jnp.float32)).reshape(1, B)


def _thresholds(bp, pri):
    return pl.pallas_call(
        _k0_body,
        out_shape=jax.ShapeDtypeStruct((1, B), jnp.float32),
    )(bp, pri)


_R = 1000


def _k3_body(x_ref, al_ref, ar_ref, w_ref, b_ref, bid_ref, thr_ref,
             nr_ref, s_ref, c_ref):
    i = pl.program_id(0)
    hl = x_ref[:, :H] + al_ref[...]
    hr = x_ref[:, H:] + ar_ref[...]
    nr = (lax.dot(hl, w_ref[:H, :], precision=_HIGH)
          + lax.dot(hr, w_ref[H:, :], precision=_HIGH) + b_ref[...])
    nr = jnp.maximum(nr, 0.0)
    nr_ref[...] = nr

    bid = bid_ref[...]
    q = bid == lax.broadcasted_iota(jnp.int32, (_R, B), 1)
    rowf = (lax.broadcasted_iota(jnp.int32, (_R, 1), 0)
            + i * _R).astype(jnp.float32)
    m = jnp.where(q & (rowf >= thr_ref[...]), 1.0, 0.0)

    s_blk = lax.dot_general(m, nr, (((0,), (0,)), ((), ())), precision=_HIGH)
    c_blk = lax.dot_general(m, jnp.ones((_R, 1), jnp.float32),
                            (((0,), (0,)), ((), ())), precision=_HIGH)

    @pl.when(i == 0)
    def _():
        s_ref[...] = jnp.zeros_like(s_ref)
        c_ref[...] = jnp.zeros_like(c_ref)
    s_ref[...] += s_blk
    c_ref[...] += c_blk


def _node_update(x, al, ar, w, bias, bid, thr):
    return pl.pallas_call(
        _k3_body,
        grid=(N // _R,),
        in_specs=[
            pl.BlockSpec((_R, D), lambda i: (i, 0)),
            pl.BlockSpec((_R, H), lambda i: (i, 0)),
            pl.BlockSpec((_R, H), lambda i: (i, 0)),
            pl.BlockSpec((D, D), lambda i: (0, 0)),
            pl.BlockSpec((1, D), lambda i: (0, 0)),
            pl.BlockSpec((_R, 1), lambda i: (i, 0)),
            pl.BlockSpec((1, B), lambda i: (0, 0)),
        ],
        out_specs=[
            pl.BlockSpec((_R, D), lambda i: (i, 0)),
            pl.BlockSpec((B, D), lambda i: (0, 0)),
            pl.BlockSpec((B, 1), lambda i: (0, 0)),
        ],
        out_shape=[
            jax.ShapeDtypeStruct((N, D), jnp.float32),
            jax.ShapeDtypeStruct((B, D), jnp.float32),
            jax.ShapeDtypeStruct((B, 1), jnp.float32),
        ],
    )(x, al, ar, w, bias, bid, thr)


def _k4_body(s_ref, c_ref, bid_ref, o_ref):
    pool = s_ref[...] / jnp.maximum(c_ref[...], 1.0)
    q = jnp.where(
        bid_ref[...] == lax.broadcasted_iota(jnp.int32, (_R, B), 1), 1.0, 0.0)
    o_ref[...] = lax.dot(q, pool, precision=_HIGH)


def _broadcast_pool(s, cnt, bid):
    return pl.pallas_call(
        _k4_body,
        grid=(N // _R,),
        in_specs=[
            pl.BlockSpec((B, D), lambda i: (0, 0)),
            pl.BlockSpec((B, 1), lambda i: (0, 0)),
            pl.BlockSpec((_R, 1), lambda i: (i, 0)),
        ],
        out_specs=pl.BlockSpec((_R, D), lambda i: (i, 0)),
        out_shape=jax.ShapeDtypeStruct((N, D), jnp.float32),
    )(s, cnt, bid)


def kernel(x, edge_index, edge_attr, batch_ids, pri_num, W_edge, W_node, b):
    src = edge_index[0]
    dst = edge_index[1]
    lo = np.asarray(_COLS_LO, np.int32)
    hi = np.asarray(_COLS_HI, np.int32)
    xl = x[:, :H]
    xr = x[:, H:]
    el, er = _edge_matmul(edge_attr,
                          W_edge[:, :H][:, lo], W_edge[:, :H][:, hi],
                          W_edge[:, H:][:, lo], W_edge[:, H:][:, hi])
    aggl, aggr = _edge_aggregate(xl, xr, src, dst,
                                 el.reshape(-1), er.reshape(-1))

    npad = 10240
    bp = jnp.full((npad,), jnp.int32(2**30)).at[:N].set(batch_ids)
    thr = _thresholds(bp.reshape(npad // 128, 128), pri_num.reshape(1, B))

    bid = batch_ids.reshape(N, 1)
    nr, s, cnt = _node_update(x, aggl, aggr, W_node, b.reshape(1, D), bid, thr)
    out2 = _broadcast_pool(s, cnt, bid)
    return jnp.concatenate([nr, out2], axis=1)


# speedup vs baseline: 1.1431x; 1.1431x over previous
import functools

import numpy as np

import jax
import jax.numpy as jnp
from jax import lax
from jax.experimental import pallas as pl
from jax.experimental.pallas import tpu as pltpu
from jax.experimental.pallas import tpu_sc as plsc

N = 10000
E = 160000
D = 256
DE = 16
B = 64

H = D // 2
NS = 16
EPT = E // NS
C = 40
NCHUNK = EPT // C
N2 = 10240
RPT = N2 // NS
RW = 40

HW = H // 2
_COLS_LO = tuple(32 * g + i for g in range(H // 32) for i in range(16))
_COLS_HI = tuple(32 * g + 16 + i for g in range(H // 32) for i in range(16))

_HIGH = lax.Precision.HIGHEST
_HI16 = np.int32(-65536)


def _pack_i32(lo_f32, hi_f32):
    lo = lax.bitcast_convert_type(lo_f32.astype(jnp.bfloat16), jnp.int16)
    hi = lax.bitcast_convert_type(hi_f32.astype(jnp.bfloat16), jnp.int16)
    return ((lo.astype(jnp.int32) & 0xFFFF)
            | (hi.astype(jnp.int32) << 16))


def _k1_body(ea_ref, wll_ref, wlh_ref, wrl_ref, wrh_ref, ol_ref, or_ref):
    a = ea_ref[...]
    ol_ref[...] = _pack_i32(lax.dot(a, wll_ref[...], precision=_HIGH),
                            lax.dot(a, wlh_ref[...], precision=_HIGH))
    or_ref[...] = _pack_i32(lax.dot(a, wrl_ref[...], precision=_HIGH),
                            lax.dot(a, wrh_ref[...], precision=_HIGH))


def _edge_matmul(ea2, wll, wlh, wrl, wrh):
    EB = 1000
    E2 = E // 2
    wspec = pl.BlockSpec((2 * DE, H), lambda i: (0, 0))
    return pl.pallas_call(
        _k1_body,
        grid=(E2 // EB,),
        in_specs=[pl.BlockSpec((EB, 2 * DE), lambda i: (i, 0)),
                  wspec, wspec, wspec, wspec],
        out_specs=[
            pl.BlockSpec((EB, H), lambda i: (i, 0)),
            pl.BlockSpec((EB, H), lambda i: (i, 0)),
        ],
        out_shape=[jax.ShapeDtypeStruct((E2, H), jnp.int32)] * 2,
    )(ea2, wll, wlh, wrl, wrh)


def _sc_body(xl, xr, src, dst, el, er, outl, outr,
             idxs0, bufx0, bufe0, bufm0, semi0, seml0, sems0,
             idxs1, bufx1, bufe1, bufm1, semi1, seml1, sems1,
             d0, d1, d2, d3, spm):
    c = lax.axis_index("c")
    s = lax.axis_index("s")
    zero16 = jnp.zeros((16,), jnp.float32)
    sets = ((idxs0, bufx0, bufe0, bufm0, semi0, seml0, sems0),
            (idxs1, bufx1, bufe1, bufm1, semi1, seml1, sems1))
    ring = (d0, d1, d2, d3)

    def _ebase(k):
        return pl.multiple_of(s * EPT + k * C, 8)

    def _stage(k, st, dd):
        eb = _ebase(k)
        pltpu.async_copy(src.at[pl.ds(eb, C)], st[0], st[4])
        pltpu.async_copy(dst.at[pl.ds(eb, C)], dd, st[4])

    def _launch(k, st):
        pltpu.make_async_copy(src.at[pl.ds(0, C)], st[0], st[4]).wait()
        pltpu.make_async_copy(dst.at[pl.ds(0, C)], st[0], st[4]).wait()
        ew_off = pl.multiple_of((s * EPT + k * C) * HW, 128)

        @pl.when(c == 0)
        def _():
            pltpu.async_copy(el.at[pl.ds(ew_off, C * HW)], st[2], st[5])
            pltpu.async_copy(xl.at[st[0]], st[1], st[5])

        @pl.when(c == 1)
        def _():
            pltpu.async_copy(er.at[pl.ds(ew_off, C * HW)], st[2], st[5])
            pltpu.async_copy(xr.at[st[0]], st[1], st[5])

    def _wait_load(st):
        pltpu.make_async_copy(el.at[pl.ds(0, C * HW)], st[2], st[5]).wait()
        pltpu.make_async_copy(xl.at[st[0]], st[1], st[5]).wait()

    def _wait_scatter(st, dd):
        pltpu.make_async_copy(st[3], spm.at[dd], st[6]).wait()

    def _zrow(i, carry):
        for p in range(H // 16):
            bufm0[i, pl.ds(p * 16, 16)] = zero16
        return carry
    lax.fori_loop(0, RW, _zrow, 0)
    base_row = s * RPT
    for j in range(RPT // RW):
        pltpu.sync_copy(bufm0, spm.at[pl.ds(base_row + j * RW, RW)])
    plsc.subcore_barrier()

    def _step(k, st_cur, st_nxt, d_cur, d_stage, first):
        if not first:
            _wait_scatter(st_cur, d_cur)
        _wait_load(st_cur)

        @pl.when(k + 1 < NCHUNK)
        def _():
            _launch(k + 1, st_nxt)

        @pl.when(k + 2 < NCHUNK)
        def _():
            _stage(k + 2, st_cur, d_stage)

        bufx, bufe, bufm = st_cur[1], st_cur[2], st_cur[3]

        def _edge(e2, cc):
            for u in range(2):
                e = e2 * 2 + u
                for g in range(H // 32):
                    we = bufe[pl.ds(e * HW + 16 * g, 16)]
                    ea = lax.bitcast_convert_type(we << 16, jnp.float32)
                    eb = lax.bitcast_convert_type(we & _HI16, jnp.float32)
                    xa = bufx[e, pl.ds(32 * g, 16)]
                    xb = bufx[e, pl.ds(32 * g + 16, 16)]
                    bufm[e, pl.ds(32 * g, 16)] = jnp.maximum(xa + ea, 0.0)
                    bufm[e, pl.ds(32 * g + 16, 16)] = jnp.maximum(xb + eb, 0.0)
            return cc
        lax.fori_loop(0, C // 2, _edge, 0)
        pltpu.async_copy(bufm, spm.at[d_cur], st_cur[6], add=True)

    _stage(0, sets[0], ring[0])
    _stage(1, sets[1], ring[1])
    _launch(0, sets[0])
    _step(0, sets[0], sets[1], ring[0], ring[2], True)
    _step(1, sets[1], sets[0], ring[1], ring[3], True)

    def _quad(j, carry):
        k = 4 * j + 2
        _step(k, sets[0], sets[1], ring[2], ring[0], False)
        _step(k + 1, sets[1], sets[0], ring[3], ring[1], False)
        _step(k + 2, sets[0], sets[1], ring[0], ring[2], False)
        _step(k + 3, sets[1], sets[0], ring[1], ring[3], False)
        return carry
    nquad = (NCHUNK - 2) // 4
    lax.fori_loop(0, nquad, _quad, 0)
    for k in range(2 + 4 * nquad, NCHUNK):
        _step(k, sets[k % 2], sets[(k + 1) % 2],
              ring[k % 4], ring[(k + 2) % 4], False)
    for k in (NCHUNK - 2, NCHUNK - 1):
        _wait_scatter(sets[k % 2], ring[k % 4])
    plsc.subcore_barrier()

    for j in range(RPT // RW):
        rs = base_row + j * RW
        pltpu.sync_copy(spm.at[pl.ds(rs, RW)], bufm0)

        @pl.when(c == 0)
        def _():
            pltpu.sync_copy(bufm0, outl.at[pl.ds(rs, RW)])

        @pl.when(c == 1)
        def _():
            pltpu.sync_copy(bufm0, outr.at[pl.ds(rs, RW)])


@functools.lru_cache(maxsize=None)
def _build_sc_kernel():
    return pl.kernel(
        _sc_body,
        mesh=plsc.VectorSubcoreMesh(
            core_axis_name="c", subcore_axis_name="s", num_cores=2,
            num_subcores=NS),
        out_type=(jax.ShapeDtypeStruct((N2, H), jnp.float32),
                  jax.ShapeDtypeStruct((N2, H), jnp.float32)),
        scratch_types=(
            [pltpu.VMEM((C,), jnp.int32),
             pltpu.VMEM((C, H), jnp.float32),
             pltpu.VMEM((C * HW,), jnp.int32),
             pltpu.VMEM((C, H), jnp.float32),
             pltpu.SemaphoreType.DMA,
             pltpu.SemaphoreType.DMA,
             pltpu.SemaphoreType.DMA] * 2
            + [pltpu.VMEM((C,), jnp.int32)] * 4
            + [pltpu.VMEM_SHARED((N2, H), jnp.float32)]),
    )


def _edge_aggregate(xl, xr, src, dst, el, er):
    return _build_sc_kernel()(xl, xr, src, dst, el, er)


def _k0_body(bp_ref, pri_ref, thr_ref):
    bi = bp_ref[...]
    starts = jnp.stack(
        [jnp.sum((bi < g).astype(jnp.float32)) for g in range(B)])
    thr_ref[...] = (starts + pri_ref[0, :].astype(jnp.float32)).reshape(1, B)


def _thresholds(bp, pri):
    return pl.pallas_call(
        _k0_body,
        out_shape=jax.ShapeDtypeStruct((1, B), jnp.float32),
    )(bp, pri)


_R = 1000


def _k3_body(x_ref, al_ref, ar_ref, w_ref, b_ref, bid_ref, thr_ref,
             nr_ref, s_ref, c_ref):
    i = pl.program_id(0)
    hl = x_ref[:, :H] + al_ref[...]
    hr = x_ref[:, H:] + ar_ref[...]
    nr = (lax.dot(hl, w_ref[:H, :], precision=_HIGH)
          + lax.dot(hr, w_ref[H:, :], precision=_HIGH) + b_ref[...])
    nr = jnp.maximum(nr, 0.0)
    nr_ref[...] = nr

    bid = bid_ref[...]
    q = bid == lax.broadcasted_iota(jnp.int32, (_R, B), 1)
    rowf = (lax.broadcasted_iota(jnp.int32, (_R, 1), 0)
            + i * _R).astype(jnp.float32)
    m = jnp.where(q & (rowf >= thr_ref[...]), 1.0, 0.0)

    s_blk = lax.dot_general(m, nr, (((0,), (0,)), ((), ())), precision=_HIGH)
    c_blk = lax.dot_general(m, jnp.ones((_R, 1), jnp.float32),
                            (((0,), (0,)), ((), ())), precision=_HIGH)

    @pl.when(i == 0)
    def _():
        s_ref[...] = jnp.zeros_like(s_ref)
        c_ref[...] = jnp.zeros_like(c_ref)
    s_ref[...] += s_blk
    c_ref[...] += c_blk


def _node_update(x, al, ar, w, bias, bid, thr):
    return pl.pallas_call(
        _k3_body,
        grid=(N // _R,),
        in_specs=[
            pl.BlockSpec((_R, D), lambda i: (i, 0)),
            pl.BlockSpec((_R, H), lambda i: (i, 0)),
            pl.BlockSpec((_R, H), lambda i: (i, 0)),
            pl.BlockSpec((D, D), lambda i: (0, 0)),
            pl.BlockSpec((1, D), lambda i: (0, 0)),
            pl.BlockSpec((_R, 1), lambda i: (i, 0)),
            pl.BlockSpec((1, B), lambda i: (0, 0)),
        ],
        out_specs=[
            pl.BlockSpec((_R, D), lambda i: (i, 0)),
            pl.BlockSpec((B, D), lambda i: (0, 0)),
            pl.BlockSpec((B, 1), lambda i: (0, 0)),
        ],
        out_shape=[
            jax.ShapeDtypeStruct((N, D), jnp.float32),
            jax.ShapeDtypeStruct((B, D), jnp.float32),
            jax.ShapeDtypeStruct((B, 1), jnp.float32),
        ],
    )(x, al, ar, w, bias, bid, thr)


def _k4_body(s_ref, c_ref, bid_ref, o_ref):
    pool = s_ref[...] / jnp.maximum(c_ref[...], 1.0)
    q = jnp.where(
        bid_ref[...] == lax.broadcasted_iota(jnp.int32, (_R, B), 1), 1.0, 0.0)
    o_ref[...] = lax.dot(q, pool, precision=_HIGH)


def _broadcast_pool(s, cnt, bid):
    return pl.pallas_call(
        _k4_body,
        grid=(N // _R,),
        in_specs=[
            pl.BlockSpec((B, D), lambda i: (0, 0)),
            pl.BlockSpec((B, 1), lambda i: (0, 0)),
            pl.BlockSpec((_R, 1), lambda i: (i, 0)),
        ],
        out_specs=pl.BlockSpec((_R, D), lambda i: (i, 0)),
        out_shape=jax.ShapeDtypeStruct((N, D), jnp.float32),
    )(s, cnt, bid)


def kernel(x, edge_index, edge_attr, batch_ids, pri_num, W_edge, W_node, b):
    src = edge_index[0]
    dst = edge_index[1]
    lo = np.asarray(_COLS_LO, np.int32)
    hi = np.asarray(_COLS_HI, np.int32)
    xl = x[:, :H]
    xr = x[:, H:]

    def _dup(w):
        z = jnp.zeros((DE, HW), w.dtype)
        return jnp.concatenate([jnp.concatenate([w, z], 1),
                                jnp.concatenate([z, w], 1)], 0)

    el, er = _edge_matmul(
        edge_attr.reshape(E // 2, 2 * DE),
        _dup(W_edge[:, :H][:, lo]), _dup(W_edge[:, :H][:, hi]),
        _dup(W_edge[:, H:][:, lo]), _dup(W_edge[:, H:][:, hi]))
    aggl, aggr = _edge_aggregate(xl, xr, src, dst,
                                 el.reshape(-1), er.reshape(-1))

    npad = 10240
    bp = jnp.full((npad,), jnp.int32(2**30)).at[:N].set(batch_ids)
    thr = _thresholds(bp.reshape(npad // 128, 128), pri_num.reshape(1, B))

    bid = batch_ids.reshape(N, 1)
    nr, s, cnt = _node_update(x, aggl, aggr, W_node, b.reshape(1, D), bid, thr)
    out2 = _broadcast_pool(s, cnt, bid)
    return jnp.concatenate([nr, out2], axis=1)


# speedup vs baseline: 1.6114x; 1.4097x over previous
import functools

import numpy as np

import jax
import jax.numpy as jnp
from jax import lax
from jax.experimental import pallas as pl
from jax.experimental.pallas import tpu as pltpu
from jax.experimental.pallas import tpu_sc as plsc

N = 10000
E = 160000
D = 256
DE = 16
B = 64

H = D // 2
NS = 16
EPT = E // NS
C = 40
NCHUNK = EPT // C
N2 = 10240
RPT = N2 // NS
RW = 40

HW = H // 2
_COLS_LO = tuple(32 * g + i for g in range(H // 32) for i in range(16))
_COLS_HI = tuple(32 * g + 16 + i for g in range(H // 32) for i in range(16))

_HIGH = lax.Precision.DEFAULT
_HI16 = np.int32(-65536)


def _pack_i32(lo_f32, hi_f32):
    lo = lax.bitcast_convert_type(lo_f32.astype(jnp.bfloat16), jnp.int16)
    hi = lax.bitcast_convert_type(hi_f32.astype(jnp.bfloat16), jnp.int16)
    return ((lo.astype(jnp.int32) & 0xFFFF)
            | (hi.astype(jnp.int32) << 16))


def _k1_body(ea_ref, wll_ref, wlh_ref, wrl_ref, wrh_ref, ol_ref, or_ref):
    a = ea_ref[...]
    ol_ref[...] = _pack_i32(lax.dot(a, wll_ref[...], precision=_HIGH),
                            lax.dot(a, wlh_ref[...], precision=_HIGH))
    or_ref[...] = _pack_i32(lax.dot(a, wrl_ref[...], precision=_HIGH),
                            lax.dot(a, wrh_ref[...], precision=_HIGH))


def _edge_matmul(ea2, wll, wlh, wrl, wrh):
    EB = 1000
    E2 = E // 2
    wspec = pl.BlockSpec((2 * DE, H), lambda i: (0, 0))
    return pl.pallas_call(
        _k1_body,
        grid=(E2 // EB,),
        in_specs=[pl.BlockSpec((EB, 2 * DE), lambda i: (i, 0)),
                  wspec, wspec, wspec, wspec],
        out_specs=[
            pl.BlockSpec((EB, H), lambda i: (i, 0)),
            pl.BlockSpec((EB, H), lambda i: (i, 0)),
        ],
        out_shape=[jax.ShapeDtypeStruct((E2, H), jnp.int32)] * 2,
    )(ea2, wll, wlh, wrl, wrh)


def _sc_body(xl, xr, src, dst, el, er, outl, outr,
             idxs0, bufx0, bufe0, bufm0, semi0, seml0, sems0,
             idxs1, bufx1, bufe1, bufm1, semi1, seml1, sems1,
             d0, d1, d2, d3, spm):
    c = lax.axis_index("c")
    s = lax.axis_index("s")
    zero16 = jnp.zeros((16,), jnp.float32)
    sets = ((idxs0, bufx0, bufe0, bufm0, semi0, seml0, sems0),
            (idxs1, bufx1, bufe1, bufm1, semi1, seml1, sems1))
    ring = (d0, d1, d2, d3)

    def _ebase(k):
        return pl.multiple_of(s * EPT + k * C, 8)

    def _stage(k, st, dd):
        eb = _ebase(k)
        pltpu.async_copy(src.at[pl.ds(eb, C)], st[0], st[4])
        pltpu.async_copy(dst.at[pl.ds(eb, C)], dd, st[4])

    def _launch(k, st):
        pltpu.make_async_copy(src.at[pl.ds(0, C)], st[0], st[4]).wait()
        pltpu.make_async_copy(dst.at[pl.ds(0, C)], st[0], st[4]).wait()
        ew_off = pl.multiple_of((s * EPT + k * C) * HW, 128)

        @pl.when(c == 0)
        def _():
            pltpu.async_copy(el.at[pl.ds(ew_off, C * HW)], st[2], st[5])
            pltpu.async_copy(xl.at[st[0]], st[1], st[5])

        @pl.when(c == 1)
        def _():
            pltpu.async_copy(er.at[pl.ds(ew_off, C * HW)], st[2], st[5])
            pltpu.async_copy(xr.at[st[0]], st[1], st[5])

    def _wait_load(st):
        pltpu.make_async_copy(el.at[pl.ds(0, C * HW)], st[2], st[5]).wait()
        pltpu.make_async_copy(xl.at[st[0]], st[1], st[5]).wait()

    def _wait_scatter(st, dd):
        pltpu.make_async_copy(st[3], spm.at[dd], st[6]).wait()

    def _zrow(i, carry):
        for p in range(H // 16):
            bufm0[i, pl.ds(p * 16, 16)] = zero16
        return carry
    lax.fori_loop(0, RW, _zrow, 0)
    base_row = s * RPT
    for j in range(RPT // RW):
        pltpu.sync_copy(bufm0, spm.at[pl.ds(base_row + j * RW, RW)])
    plsc.subcore_barrier()

    def _step(k, st_cur, st_nxt, d_cur, d_stage, first):
        if not first:
            _wait_scatter(st_cur, d_cur)
        _wait_load(st_cur)

        @pl.when(k + 1 < NCHUNK)
        def _():
            _launch(k + 1, st_nxt)

        @pl.when(k + 2 < NCHUNK)
        def _():
            _stage(k + 2, st_cur, d_stage)

        bufx, bufe, bufm = st_cur[1], st_cur[2], st_cur[3]

        def _edge(e2, cc):
            for u in range(2):
                e = e2 * 2 + u
                for g in range(H // 32):
                    we = bufe[pl.ds(e * HW + 16 * g, 16)]
                    ea = lax.bitcast_convert_type(we << 16, jnp.float32)
                    eb = lax.bitcast_convert_type(we & _HI16, jnp.float32)
                    xa = bufx[e, pl.ds(32 * g, 16)]
                    xb = bufx[e, pl.ds(32 * g + 16, 16)]
                    bufm[e, pl.ds(32 * g, 16)] = jnp.maximum(xa + ea, 0.0)
                    bufm[e, pl.ds(32 * g + 16, 16)] = jnp.maximum(xb + eb, 0.0)
            return cc
        lax.fori_loop(0, C // 2, _edge, 0)
        pltpu.async_copy(bufm, spm.at[d_cur], st_cur[6], add=True)

    _stage(0, sets[0], ring[0])
    _stage(1, sets[1], ring[1])
    _launch(0, sets[0])
    _step(0, sets[0], sets[1], ring[0], ring[2], True)
    _step(1, sets[1], sets[0], ring[1], ring[3], True)

    def _quad(j, carry):
        k = 4 * j + 2
        _step(k, sets[0], sets[1], ring[2], ring[0], False)
        _step(k + 1, sets[1], sets[0], ring[3], ring[1], False)
        _step(k + 2, sets[0], sets[1], ring[0], ring[2], False)
        _step(k + 3, sets[1], sets[0], ring[1], ring[3], False)
        return carry
    nquad = (NCHUNK - 2) // 4
    lax.fori_loop(0, nquad, _quad, 0)
    for k in range(2 + 4 * nquad, NCHUNK):
        _step(k, sets[k % 2], sets[(k + 1) % 2],
              ring[k % 4], ring[(k + 2) % 4], False)
    for k in (NCHUNK - 2, NCHUNK - 1):
        _wait_scatter(sets[k % 2], ring[k % 4])
    plsc.subcore_barrier()

    for j in range(RPT // RW):
        rs = base_row + j * RW
        pltpu.sync_copy(spm.at[pl.ds(rs, RW)], bufm0)

        @pl.when(c == 0)
        def _():
            pltpu.sync_copy(bufm0, outl.at[pl.ds(rs, RW)])

        @pl.when(c == 1)
        def _():
            pltpu.sync_copy(bufm0, outr.at[pl.ds(rs, RW)])


@functools.lru_cache(maxsize=None)
def _build_sc_kernel():
    return pl.kernel(
        _sc_body,
        mesh=plsc.VectorSubcoreMesh(
            core_axis_name="c", subcore_axis_name="s", num_cores=2,
            num_subcores=NS),
        out_type=(jax.ShapeDtypeStruct((N2, H), jnp.float32),
                  jax.ShapeDtypeStruct((N2, H), jnp.float32)),
        scratch_types=(
            [pltpu.VMEM((C,), jnp.int32),
             pltpu.VMEM((C, H), jnp.float32),
             pltpu.VMEM((C * HW,), jnp.int32),
             pltpu.VMEM((C, H), jnp.float32),
             pltpu.SemaphoreType.DMA,
             pltpu.SemaphoreType.DMA,
             pltpu.SemaphoreType.DMA] * 2
            + [pltpu.VMEM((C,), jnp.int32)] * 4
            + [pltpu.VMEM_SHARED((N2, H), jnp.float32)]),
    )


def _edge_aggregate(xl, xr, src, dst, el, er):
    return _build_sc_kernel()(xl, xr, src, dst, el, er)


def _k0_body(bp_ref, pri_ref, thr_ref):
    bi = bp_ref[...]
    starts = jnp.stack(
        [jnp.sum((bi < g).astype(jnp.float32)) for g in range(B)])
    thr_ref[...] = (starts + pri_ref[0, :].astype(jnp.float32)).reshape(1, B)


def _thresholds(bp, pri):
    return pl.pallas_call(
        _k0_body,
        out_shape=jax.ShapeDtypeStruct((1, B), jnp.float32),
    )(bp, pri)


_R = 1000


def _k3_body(x_ref, al_ref, ar_ref, w_ref, b_ref, bid_ref, thr_ref,
             nr_ref, s_ref, c_ref):
    i = pl.program_id(0)
    hl = x_ref[:, :H] + al_ref[...]
    hr = x_ref[:, H:] + ar_ref[...]
    nr = (lax.dot(hl, w_ref[:H, :], precision=_HIGH)
          + lax.dot(hr, w_ref[H:, :], precision=_HIGH) + b_ref[...])
    nr = jnp.maximum(nr, 0.0)
    nr_ref[...] = nr

    bid = bid_ref[...]
    q = bid == lax.broadcasted_iota(jnp.int32, (_R, B), 1)
    rowf = (lax.broadcasted_iota(jnp.int32, (_R, 1), 0)
            + i * _R).astype(jnp.float32)
    m = jnp.where(q & (rowf >= thr_ref[...]), 1.0, 0.0)

    s_blk = lax.dot_general(m, nr, (((0,), (0,)), ((), ())), precision=_HIGH)
    c_blk = lax.dot_general(m, jnp.ones((_R, 1), jnp.float32),
                            (((0,), (0,)), ((), ())), precision=_HIGH)

    @pl.when(i == 0)
    def _():
        s_ref[...] = jnp.zeros_like(s_ref)
        c_ref[...] = jnp.zeros_like(c_ref)
    s_ref[...] += s_blk
    c_ref[...] += c_blk


def _node_update(x, al, ar, w, bias, bid, thr):
    return pl.pallas_call(
        _k3_body,
        grid=(N // _R,),
        in_specs=[
            pl.BlockSpec((_R, D), lambda i: (i, 0)),
            pl.BlockSpec((_R, H), lambda i: (i, 0)),
            pl.BlockSpec((_R, H), lambda i: (i, 0)),
            pl.BlockSpec((D, D), lambda i: (0, 0)),
            pl.BlockSpec((1, D), lambda i: (0, 0)),
            pl.BlockSpec((_R, 1), lambda i: (i, 0)),
            pl.BlockSpec((1, B), lambda i: (0, 0)),
        ],
        out_specs=[
            pl.BlockSpec((_R, D), lambda i: (i, 0)),
            pl.BlockSpec((B, D), lambda i: (0, 0)),
            pl.BlockSpec((B, 1), lambda i: (0, 0)),
        ],
        out_shape=[
            jax.ShapeDtypeStruct((N, D), jnp.float32),
            jax.ShapeDtypeStruct((B, D), jnp.float32),
            jax.ShapeDtypeStruct((B, 1), jnp.float32),
        ],
    )(x, al, ar, w, bias, bid, thr)


def _k4_body(s_ref, c_ref, bid_ref, o_ref):
    pool = s_ref[...] / jnp.maximum(c_ref[...], 1.0)
    q = jnp.where(
        bid_ref[...] == lax.broadcasted_iota(jnp.int32, (_R, B), 1), 1.0, 0.0)
    o_ref[...] = lax.dot(q, pool, precision=_HIGH)


def _broadcast_pool(s, cnt, bid):
    return pl.pallas_call(
        _k4_body,
        grid=(N // _R,),
        in_specs=[
            pl.BlockSpec((B, D), lambda i: (0, 0)),
            pl.BlockSpec((B, 1), lambda i: (0, 0)),
            pl.BlockSpec((_R, 1), lambda i: (i, 0)),
        ],
        out_specs=pl.BlockSpec((_R, D), lambda i: (i, 0)),
        out_shape=jax.ShapeDtypeStruct((N, D), jnp.float32),
    )(s, cnt, bid)


def kernel(x, edge_index, edge_attr, batch_ids, pri_num, W_edge, W_node, b):
    src = edge_index[0]
    dst = edge_index[1]
    lo = np.asarray(_COLS_LO, np.int32)
    hi = np.asarray(_COLS_HI, np.int32)
    xl = x[:, :H]
    xr = x[:, H:]

    def _dup(w):
        z = jnp.zeros((DE, HW), w.dtype)
        return jnp.concatenate([jnp.concatenate([w, z], 1),
                                jnp.concatenate([z, w], 1)], 0)

    el, er = _edge_matmul(
        edge_attr.reshape(E // 2, 2 * DE),
        _dup(W_edge[:, :H][:, lo]), _dup(W_edge[:, :H][:, hi]),
        _dup(W_edge[:, H:][:, lo]), _dup(W_edge[:, H:][:, hi]))
    aggl, aggr = _edge_aggregate(xl, xr, src, dst,
                                 el.reshape(-1), er.reshape(-1))

    npad = 10240
    bp = jnp.full((npad,), jnp.int32(2**30)).at[:N].set(batch_ids)
    thr = _thresholds(bp.reshape(npad // 128, 128), pri_num.reshape(1, B))

    bid = batch_ids.reshape(N, 1)
    nr, s, cnt = _node_update(x, aggl, aggr, W_node, b.reshape(1, D), bid, thr)
    out2 = _broadcast_pool(s, cnt, bid)
    return jnp.concatenate([nr, out2], axis=1)
